# baseline (speedup 1.0000x reference)
import functools

import jax
import jax.numpy as jnp
from jax import lax
from jax.experimental import pallas as pl
from jax.experimental.pallas import tpu as pltpu
from jax.experimental.pallas import tpu_sc as plsc

NC = 2
NS = 16
L = 16
NW = NC * NS


def kernel(x, pos_embedding, token_embedding):
    B, S = x.shape
    V, D = token_embedding.shape
    CH = S // NW

    xi = x.astype(jnp.int32)

    mesh = plsc.VectorSubcoreMesh(core_axis_name="c", subcore_axis_name="s")

    @functools.partial(
        pl.kernel,
        out_type=jax.ShapeDtypeStruct((B, S, D), jnp.float32),
        mesh=mesh,
        scratch_types=[
            pltpu.VMEM((B, CH), jnp.int32),
            pltpu.VMEM((CH, D), jnp.float32),
            pltpu.VMEM((CH, D), jnp.float32),
            pltpu.VMEM((CH, D), jnp.float32),
            pltpu.SemaphoreType.DMA,
            pltpu.SemaphoreType.DMA,
            pltpu.SemaphoreType.DMA,
            pltpu.SemaphoreType.DMA,
            pltpu.SemaphoreType.DMA,
        ],
    )
    def run(x_hbm, pos_hbm, tok_hbm, out_hbm,
            idx_v, pos_v, tok0, tok1, g0, g1, st0, st1, psem):
        wid = lax.axis_index("s") * NC + lax.axis_index("c")
        pbase = wid * CH

        for b in range(B):
            pltpu.sync_copy(x_hbm.at[b, pl.ds(pbase, CH)], idx_v.at[b])
        toks = [tok0, tok1]
        gsems = [g0, g1]
        ssems = [st0, st1]

        gcp = {0: pltpu.async_copy(tok_hbm.at[idx_v.at[0]], toks[0], gsems[0])}
        pltpu.async_copy(pos_hbm.at[pl.ds(pbase, CH)], pos_v, psem).wait()

        stcp = {}
        for b in range(B):
            cur = b % 2
            if b + 1 < B:
                if b >= 1:
                    stcp[b - 1].wait()
                gcp[b + 1] = pltpu.async_copy(
                    tok_hbm.at[idx_v.at[b + 1]], toks[1 - cur], gsems[1 - cur])
            gcp[b].wait()

            tok_v = toks[cur]

            def row_add(r, carry):
                for k in range(D // L):
                    sl = pl.ds(k * L, L)
                    tok_v[r, sl] = tok_v[r, sl] + pos_v[r, sl]
                return carry

            lax.fori_loop(0, CH, row_add, 0)

            stcp[b] = pltpu.async_copy(
                tok_v, out_hbm.at[b, pl.ds(pbase, CH)], ssems[cur])

        stcp[B - 2].wait()
        stcp[B - 1].wait()

    return run(xi, pos_embedding, token_embedding)


# speedup vs baseline: 1.0484x; 1.0484x over previous
import functools

import jax
import jax.numpy as jnp
from jax import lax
from jax.experimental import pallas as pl
from jax.experimental.pallas import tpu as pltpu
from jax.experimental.pallas import tpu_sc as plsc

NC = 2
NS = 16
L = 16
NW = NC * NS


def kernel(x, pos_embedding, token_embedding):
    B, S = x.shape
    V, D = token_embedding.shape
    CH = S // NW

    xi = x.astype(jnp.int32)

    mesh = plsc.VectorSubcoreMesh(core_axis_name="c", subcore_axis_name="s")

    @functools.partial(
        pl.kernel,
        out_type=jax.ShapeDtypeStruct((B, S, D), jnp.float32),
        mesh=mesh,
        scratch_types=[
            pltpu.VMEM((B, CH), jnp.int32),
            pltpu.VMEM((CH, D), jnp.float32),
            pltpu.VMEM((CH, D), jnp.float32),
            pltpu.VMEM((CH, D), jnp.float32),
            pltpu.SemaphoreType.DMA,
            pltpu.SemaphoreType.DMA,
            pltpu.SemaphoreType.DMA,
            pltpu.SemaphoreType.DMA,
            pltpu.SemaphoreType.DMA,
        ],
    )
    def run(x_hbm, pos_hbm, tok_hbm, out_hbm,
            idx_v, pos_v, tok0, tok1, g0, g1, st0, st1, psem):
        wid = lax.axis_index("s") * NC + lax.axis_index("c")
        pbase = wid * CH

        pltpu.sync_copy(x_hbm.at[:, pl.ds(pbase, CH)], idx_v)
        toks = [tok0, tok1]
        gsems = [g0, g1]
        ssems = [st0, st1]

        gcp = {0: pltpu.async_copy(tok_hbm.at[idx_v.at[0]], toks[0], gsems[0])}
        pltpu.async_copy(pos_hbm.at[pl.ds(pbase, CH)], pos_v, psem).wait()

        stcp = {}
        for b in range(B):
            cur = b % 2
            if b + 1 < B:
                if b >= 1:
                    stcp[b - 1].wait()
                gcp[b + 1] = pltpu.async_copy(
                    tok_hbm.at[idx_v.at[b + 1]], toks[1 - cur], gsems[1 - cur])
            gcp[b].wait()

            tok_v = toks[cur]

            def row_add(r, carry):
                for k in range(D // L):
                    sl = pl.ds(k * L, L)
                    tok_v[r, sl] = tok_v[r, sl] + pos_v[r, sl]
                return carry

            lax.fori_loop(0, CH, row_add, 0)

            stcp[b] = pltpu.async_copy(
                tok_v, out_hbm.at[b, pl.ds(pbase, CH)], ssems[cur])

        stcp[B - 2].wait()
        stcp[B - 1].wait()

    return run(xi, pos_embedding, token_embedding)
